# baseline (speedup 1.0000x reference)
import functools
import math

import jax
import jax.numpy as jnp
from jax import lax
from jax.experimental import pallas as pl
from jax.experimental.pallas import tpu as pltpu
from jax.experimental.pallas import tpu_sc as plsc

ABS_MAX = 1000000.0
SCALE = 0.1
N = 10000
D = 128
E = 320000

NC = 2
NS = 16
NW = NC * NS

CHUNK = 128
CPT0 = 94
CPT1 = 64
TOTAL_CHUNKS = NS * (CPT0 + CPT1)
TOTAL_ALLOC = TOTAL_CHUNKS + 2
ROWS_PAD = 10240
RPT = ROWS_PAD // NS

_mesh = plsc.VectorSubcoreMesh(
    core_axis_name="c", subcore_axis_name="s", num_cores=NC, num_subcores=NS)


@functools.partial(
    pl.kernel,
    out_type=jax.ShapeDtypeStruct((NC, ROWS_PAD, D), jnp.float32),
    mesh=_mesh,
    scratch_types=[
        pltpu.VMEM((2, 2, CHUNK), jnp.int32),
        pltpu.VMEM((2, CHUNK, D), jnp.float32),
        pltpu.VMEM_SHARED((ROWS_PAD, D), jnp.float32),
        pltpu.SemaphoreType.DMA,
        pltpu.SemaphoreType.DMA,
    ],
)
def _sc_scatter(feat_hbm, sd_hbm, out_hbm, idx_v, rows_v, acc_sh, sem0, sem1):
    c = lax.axis_index("c")
    s = lax.axis_index("s")

    with jax.named_scope("zero_acc"):
        zeros16 = jnp.zeros((16,), jnp.float32)

        def zfill_body(r, _):
            for j in range(D // 16):
                rows_v[0, r, pl.ds(j * 16, 16)] = zeros16
            return 0
        lax.fori_loop(0, CHUNK, zfill_body, 0)

        def zero_body(k, _):
            pltpu.sync_copy(rows_v.at[0],
                            acc_sh.at[pl.ds(s * RPT + k * CHUNK, CHUNK)])
            return 0
        lax.fori_loop(0, RPT // CHUNK, zero_body, 0)

        plsc.subcore_barrier()

    with jax.named_scope("edge_loop"):
        base = lax.select(c == 0, s * CPT0, NS * CPT0 + s * CPT1)
        cpt = lax.select(c == 0, CPT0, CPT1)
        sems = (sem0, sem1)

        for b in range(2):
            pltpu.sync_copy(sd_hbm.at[base + b], idx_v.at[b])
            pltpu.async_copy(feat_hbm.at[idx_v.at[b, 0]], rows_v.at[b],
                             sems[b])

        def chunk_body(k, _):
            for b in range(2):
                i = 2 * k + b
                pltpu.make_async_copy(
                    feat_hbm.at[pl.ds(0, CHUNK)], rows_v.at[b],
                    sems[b]).wait()
                pltpu.sync_copy(rows_v.at[b], acc_sh.at[idx_v.at[b, 1]],
                                add=True)
                pltpu.sync_copy(sd_hbm.at[base + i + 2], idx_v.at[b])
                pltpu.async_copy(feat_hbm.at[idx_v.at[b, 0]], rows_v.at[b],
                                 sems[b])
            return 0
        lax.fori_loop(0, cpt // 2, chunk_body, 0)

        for b in range(2):
            pltpu.make_async_copy(
                feat_hbm.at[pl.ds(0, CHUNK)], rows_v.at[b], sems[b]).wait()

        plsc.subcore_barrier()

    with jax.named_scope("copy_out"):
        pltpu.sync_copy(acc_sh.at[pl.ds(s * RPT, RPT)],
                        out_hbm.at[c, pl.ds(s * RPT, RPT)])


def _tc_body(h_ref, f_ref, r_ref, o_ref):
    hsum = (h_ref[0] + h_ref[1]) * jnp.float32(SCALE)
    h = jnp.clip(hsum, -ABS_MAX, ABS_MAX)
    fe = jnp.clip(f_ref[...] * jnp.float32(SCALE), -ABS_MAX, ABS_MAX)
    feats = jnp.clip(h + fe, -ABS_MAX, ABS_MAX)
    o_ref[...] = lax.dot(feats, r_ref[...],
                         precision=lax.Precision.HIGHEST,
                         preferred_element_type=jnp.float32)


_BR = 1000

_tc_project = pl.pallas_call(
    _tc_body,
    grid=(N // _BR,),
    in_specs=[
        pl.BlockSpec((NC, _BR, D), lambda i: (0, i, 0)),
        pl.BlockSpec((_BR, D), lambda i: (i, 0)),
        pl.BlockSpec((D, D), lambda i: (0, 0)),
    ],
    out_specs=pl.BlockSpec((_BR, D), lambda i: (i, 0)),
    out_shape=jax.ShapeDtypeStruct((N, D), jnp.float32),
)


def kernel(features, edge_index, R):
    src = edge_index[0].astype(jnp.int32)
    dst = edge_index[1].astype(jnp.int32)
    live_pad = TOTAL_ALLOC * CHUNK - E
    src3 = jnp.concatenate([src, jnp.zeros((live_pad,), jnp.int32)])
    dst3 = jnp.concatenate([dst, jnp.full((live_pad,), N, jnp.int32)])
    sd = jnp.stack([src3.reshape(TOTAL_ALLOC, CHUNK),
                    dst3.reshape(TOTAL_ALLOC, CHUNK)], axis=1)
    h2 = _sc_scatter(features, sd)
    r_scaled = R * jnp.float32(1.0 / (math.sqrt(D) * math.sqrt(D)))
    return _tc_project(h2, features, r_scaled)


# speedup vs baseline: 1.0997x; 1.0997x over previous
import functools
import math

import jax
import jax.numpy as jnp
from jax import lax
from jax.experimental import pallas as pl
from jax.experimental.pallas import tpu as pltpu
from jax.experimental.pallas import tpu_sc as plsc

ABS_MAX = 1000000.0
SCALE = 0.1
N = 10000
D = 128
E = 320000

NC = 2
NS = 16
NW = NC * NS

CHUNK = 128
CPT0 = 118
CPT1 = 40
TOTAL_CHUNKS = NS * (CPT0 + CPT1)
TOTAL_ALLOC = TOTAL_CHUNKS + 2
ROWS_PAD = 10240
RPT = ROWS_PAD // NS

_mesh = plsc.VectorSubcoreMesh(
    core_axis_name="c", subcore_axis_name="s", num_cores=NC, num_subcores=NS)


@functools.partial(
    pl.kernel,
    out_type=jax.ShapeDtypeStruct((NC, ROWS_PAD, D), jnp.float32),
    mesh=_mesh,
    scratch_types=[
        pltpu.VMEM((2, 2, CHUNK), jnp.int32),
        pltpu.VMEM((2, CHUNK, D), jnp.float32),
        pltpu.VMEM_SHARED((ROWS_PAD, D), jnp.float32),
        pltpu.SemaphoreType.DMA,
        pltpu.SemaphoreType.DMA,
    ],
)
def _sc_scatter(feat_hbm, sd_hbm, out_hbm, idx_v, rows_v, acc_sh, sem0, sem1):
    c = lax.axis_index("c")
    s = lax.axis_index("s")

    with jax.named_scope("zero_acc"):
        zeros16 = jnp.zeros((16,), jnp.float32)

        def zfill_body(r, _):
            for j in range(D // 16):
                rows_v[0, r, pl.ds(j * 16, 16)] = zeros16
            return 0
        lax.fori_loop(0, CHUNK, zfill_body, 0)

        def zero_body(k, _):
            pltpu.sync_copy(rows_v.at[0],
                            acc_sh.at[pl.ds(s * RPT + k * CHUNK, CHUNK)])
            return 0
        lax.fori_loop(0, RPT // CHUNK, zero_body, 0)

        plsc.subcore_barrier()

    with jax.named_scope("edge_loop"):
        base = lax.select(c == 0, s * CPT0, NS * CPT0 + s * CPT1)
        cpt = lax.select(c == 0, CPT0, CPT1)
        sems = (sem0, sem1)

        for b in range(2):
            pltpu.sync_copy(sd_hbm.at[base + b], idx_v.at[b])
            pltpu.async_copy(feat_hbm.at[idx_v.at[b, 0]], rows_v.at[b],
                             sems[b])

        def chunk_body(k, _):
            for b in range(2):
                i = 2 * k + b
                pltpu.make_async_copy(
                    feat_hbm.at[pl.ds(0, CHUNK)], rows_v.at[b],
                    sems[b]).wait()
                pltpu.sync_copy(rows_v.at[b], acc_sh.at[idx_v.at[b, 1]],
                                add=True)
                pltpu.sync_copy(sd_hbm.at[base + i + 2], idx_v.at[b])
                pltpu.async_copy(feat_hbm.at[idx_v.at[b, 0]], rows_v.at[b],
                                 sems[b])
            return 0
        lax.fori_loop(0, cpt // 2, chunk_body, 0)

        for b in range(2):
            pltpu.make_async_copy(
                feat_hbm.at[pl.ds(0, CHUNK)], rows_v.at[b], sems[b]).wait()

        plsc.subcore_barrier()

    with jax.named_scope("copy_out"):
        pltpu.sync_copy(acc_sh.at[pl.ds(s * RPT, RPT)],
                        out_hbm.at[c, pl.ds(s * RPT, RPT)])


def _tc_body(h_ref, f_ref, r_ref, o_ref):
    hsum = (h_ref[0] + h_ref[1]) * jnp.float32(SCALE)
    h = jnp.clip(hsum, -ABS_MAX, ABS_MAX)
    fe = jnp.clip(f_ref[...] * jnp.float32(SCALE), -ABS_MAX, ABS_MAX)
    feats = jnp.clip(h + fe, -ABS_MAX, ABS_MAX)
    o_ref[...] = lax.dot(feats, r_ref[...],
                         precision=lax.Precision.HIGHEST,
                         preferred_element_type=jnp.float32)


_BR = 1000

_tc_project = pl.pallas_call(
    _tc_body,
    grid=(N // _BR,),
    in_specs=[
        pl.BlockSpec((NC, _BR, D), lambda i: (0, i, 0)),
        pl.BlockSpec((_BR, D), lambda i: (i, 0)),
        pl.BlockSpec((D, D), lambda i: (0, 0)),
    ],
    out_specs=pl.BlockSpec((_BR, D), lambda i: (i, 0)),
    out_shape=jax.ShapeDtypeStruct((N, D), jnp.float32),
)


def kernel(features, edge_index, R):
    src = edge_index[0].astype(jnp.int32)
    dst = edge_index[1].astype(jnp.int32)
    live_pad = TOTAL_ALLOC * CHUNK - E
    src3 = jnp.concatenate([src, jnp.zeros((live_pad,), jnp.int32)])
    dst3 = jnp.concatenate([dst, jnp.full((live_pad,), N, jnp.int32)])
    sd = jnp.stack([src3.reshape(TOTAL_ALLOC, CHUNK),
                    dst3.reshape(TOTAL_ALLOC, CHUNK)], axis=1)
    h2 = _sc_scatter(features, sd)
    r_scaled = R * jnp.float32(1.0 / (math.sqrt(D) * math.sqrt(D)))
    return _tc_project(h2, features, r_scaled)


# speedup vs baseline: 1.3935x; 1.2672x over previous
import functools
import math

import jax
import jax.numpy as jnp
from jax import lax
from jax.experimental import pallas as pl
from jax.experimental.pallas import tpu as pltpu
from jax.experimental.pallas import tpu_sc as plsc

ABS_MAX = 1000000.0
SCALE = 0.1
N = 10000
D = 128
E = 320000

NC = 2
NS = 16
NW = NC * NS

CHUNK = 128
CPT0 = 114
CPT1 = 43
TOTAL_CHUNKS = NS * (CPT0 + CPT1)
TOTAL_ALLOC = TOTAL_CHUNKS + 2
ROWS_PAD = 10240
RPT = ROWS_PAD // NS

_mesh = plsc.VectorSubcoreMesh(
    core_axis_name="c", subcore_axis_name="s", num_cores=NC, num_subcores=NS)


@functools.partial(
    pl.kernel,
    out_type=jax.ShapeDtypeStruct((NC, ROWS_PAD, D), jnp.float32),
    mesh=_mesh,
    scratch_types=[
        pltpu.VMEM((2, 2, CHUNK), jnp.int32),
        pltpu.VMEM((2, CHUNK, D), jnp.float32),
        pltpu.VMEM_SHARED((ROWS_PAD, D), jnp.float32),
        pltpu.SemaphoreType.DMA,
        pltpu.SemaphoreType.DMA,
    ],
)
def _sc_scatter(feat_hbm, sd_hbm, out_hbm, idx_v, rows_v, acc_sh, sem0, sem1):
    c = lax.axis_index("c")
    s = lax.axis_index("s")

    with jax.named_scope("zero_acc"):
        zeros16 = jnp.zeros((16,), jnp.float32)

        def zfill_body(r, _):
            for j in range(D // 16):
                rows_v[0, r, pl.ds(j * 16, 16)] = zeros16
            return 0
        lax.fori_loop(0, CHUNK, zfill_body, 0)

        def zero_body(k, _):
            pltpu.sync_copy(rows_v.at[0],
                            acc_sh.at[pl.ds(s * RPT + k * CHUNK, CHUNK)])
            return 0
        lax.fori_loop(0, RPT // CHUNK, zero_body, 0)

        plsc.subcore_barrier()

    with jax.named_scope("edge_loop"):
        sems = (sem0, sem1)

        @pl.when(c == 0)
        def _core0_ring():
            base = s * CPT0
            for b in range(2):
                pltpu.sync_copy(sd_hbm.at[base + b], idx_v.at[b])
                pltpu.async_copy(feat_hbm.at[idx_v.at[b, 0]], rows_v.at[b],
                                 sems[b])

            def chunk_body(k, _):
                for b in range(2):
                    i = 2 * k + b
                    pltpu.make_async_copy(
                        feat_hbm.at[pl.ds(0, CHUNK)], rows_v.at[b],
                        sems[b]).wait()
                    pltpu.sync_copy(rows_v.at[b], acc_sh.at[idx_v.at[b, 1]],
                                    add=True)
                    pltpu.sync_copy(sd_hbm.at[base + i + 2], idx_v.at[b])
                    pltpu.async_copy(feat_hbm.at[idx_v.at[b, 0]],
                                     rows_v.at[b], sems[b])
                return 0
            lax.fori_loop(0, CPT0 // 2, chunk_body, 0)

            for b in range(2):
                pltpu.make_async_copy(
                    feat_hbm.at[pl.ds(0, CHUNK)], rows_v.at[b],
                    sems[b]).wait()

        @pl.when(c != 0)
        def _core1_serial():
            base = NS * CPT0 + s * CPT1

            def chunk_body(i, _):
                pltpu.sync_copy(sd_hbm.at[base + i], idx_v.at[0])
                pltpu.async_copy(feat_hbm.at[idx_v.at[0, 0]], rows_v.at[0],
                                 sem0).wait()
                pltpu.sync_copy(rows_v.at[0], acc_sh.at[idx_v.at[0, 1]],
                                add=True)
                return 0
            lax.fori_loop(0, CPT1, chunk_body, 0)

        plsc.subcore_barrier()

    with jax.named_scope("copy_out"):
        pltpu.sync_copy(acc_sh.at[pl.ds(s * RPT, RPT)],
                        out_hbm.at[c, pl.ds(s * RPT, RPT)])


def _tc_body(h_ref, f_ref, r_ref, o_ref):
    hsum = (h_ref[0] + h_ref[1]) * jnp.float32(SCALE)
    h = jnp.clip(hsum, -ABS_MAX, ABS_MAX)
    fe = jnp.clip(f_ref[...] * jnp.float32(SCALE), -ABS_MAX, ABS_MAX)
    feats = jnp.clip(h + fe, -ABS_MAX, ABS_MAX)
    o_ref[...] = lax.dot(feats, r_ref[...],
                         precision=lax.Precision.HIGHEST,
                         preferred_element_type=jnp.float32)


_BR = 1000

_tc_project = pl.pallas_call(
    _tc_body,
    grid=(N // _BR,),
    in_specs=[
        pl.BlockSpec((NC, _BR, D), lambda i: (0, i, 0)),
        pl.BlockSpec((_BR, D), lambda i: (i, 0)),
        pl.BlockSpec((D, D), lambda i: (0, 0)),
    ],
    out_specs=pl.BlockSpec((_BR, D), lambda i: (i, 0)),
    out_shape=jax.ShapeDtypeStruct((N, D), jnp.float32),
)


def kernel(features, edge_index, R):
    src = edge_index[0].astype(jnp.int32)
    dst = edge_index[1].astype(jnp.int32)
    live_pad = TOTAL_ALLOC * CHUNK - E
    src3 = jnp.concatenate([src, jnp.zeros((live_pad,), jnp.int32)])
    dst3 = jnp.concatenate([dst, jnp.full((live_pad,), N, jnp.int32)])
    sd = jnp.stack([src3.reshape(TOTAL_ALLOC, CHUNK),
                    dst3.reshape(TOTAL_ALLOC, CHUNK)], axis=1)
    h2 = _sc_scatter(features, sd)
    r_scaled = R * jnp.float32(1.0 / (math.sqrt(D) * math.sqrt(D)))
    return _tc_project(h2, features, r_scaled)


# speedup vs baseline: 1.4357x; 1.0303x over previous
import functools
import math

import jax
import jax.numpy as jnp
from jax import lax
from jax.experimental import pallas as pl
from jax.experimental.pallas import tpu as pltpu
from jax.experimental.pallas import tpu_sc as plsc

ABS_MAX = 1000000.0
SCALE = 0.1
N = 10000
D = 128
E = 320000

NC = 2
NS = 16
NW = NC * NS

CHUNK = 128
CPT0 = 118
CPT1 = 39
TOTAL_CHUNKS = NS * (CPT0 + CPT1)
TOTAL_ALLOC = TOTAL_CHUNKS + 2
ROWS_PAD = 10240
RPT = ROWS_PAD // NS

_mesh = plsc.VectorSubcoreMesh(
    core_axis_name="c", subcore_axis_name="s", num_cores=NC, num_subcores=NS)


@functools.partial(
    pl.kernel,
    out_type=jax.ShapeDtypeStruct((NC, ROWS_PAD, D), jnp.float32),
    mesh=_mesh,
    scratch_types=[
        pltpu.VMEM((2, 2, CHUNK), jnp.int32),
        pltpu.VMEM((2, CHUNK, D), jnp.float32),
        pltpu.VMEM_SHARED((ROWS_PAD, D), jnp.float32),
        pltpu.SemaphoreType.DMA,
        pltpu.SemaphoreType.DMA,
    ],
)
def _sc_scatter(feat_hbm, sd_hbm, out_hbm, idx_v, rows_v, acc_sh, sem0, sem1):
    c = lax.axis_index("c")
    s = lax.axis_index("s")

    with jax.named_scope("zero_acc"):
        zeros16 = jnp.zeros((16,), jnp.float32)

        def zfill_body(r, _):
            for j in range(D // 16):
                rows_v[0, r, pl.ds(j * 16, 16)] = zeros16
            return 0
        lax.fori_loop(0, CHUNK, zfill_body, 0)

        def zero_body(k, _):
            pltpu.sync_copy(rows_v.at[0],
                            acc_sh.at[pl.ds(s * RPT + k * CHUNK, CHUNK)])
            return 0
        lax.fori_loop(0, RPT // CHUNK, zero_body, 0)

        plsc.subcore_barrier()

    with jax.named_scope("edge_loop"):
        sems = (sem0, sem1)

        @pl.when(c == 0)
        def _core0_ring():
            base = s * CPT0
            for b in range(2):
                pltpu.sync_copy(sd_hbm.at[base + b], idx_v.at[b])
                pltpu.async_copy(feat_hbm.at[idx_v.at[b, 0]], rows_v.at[b],
                                 sems[b])

            def chunk_body(k, _):
                for b in range(2):
                    i = 2 * k + b
                    pltpu.make_async_copy(
                        feat_hbm.at[pl.ds(0, CHUNK)], rows_v.at[b],
                        sems[b]).wait()
                    pltpu.sync_copy(rows_v.at[b], acc_sh.at[idx_v.at[b, 1]],
                                    add=True)
                    pltpu.sync_copy(sd_hbm.at[base + i + 2], idx_v.at[b])
                    pltpu.async_copy(feat_hbm.at[idx_v.at[b, 0]],
                                     rows_v.at[b], sems[b])
                return 0
            lax.fori_loop(0, CPT0 // 2, chunk_body, 0)

            for b in range(2):
                pltpu.make_async_copy(
                    feat_hbm.at[pl.ds(0, CHUNK)], rows_v.at[b],
                    sems[b]).wait()

        @pl.when(c != 0)
        def _core1_serial():
            base = NS * CPT0 + s * CPT1

            def chunk_body(i, _):
                pltpu.sync_copy(sd_hbm.at[base + i], idx_v.at[0])
                pltpu.async_copy(feat_hbm.at[idx_v.at[0, 0]], rows_v.at[0],
                                 sem0).wait()
                pltpu.sync_copy(rows_v.at[0], acc_sh.at[idx_v.at[0, 1]],
                                add=True)
                return 0
            lax.fori_loop(0, CPT1, chunk_body, 0)

        plsc.subcore_barrier()

    with jax.named_scope("copy_out"):
        pltpu.sync_copy(acc_sh.at[pl.ds(s * RPT, RPT)],
                        out_hbm.at[c, pl.ds(s * RPT, RPT)])


def _tc_body(h_ref, f_ref, r_ref, o_ref):
    hsum = (h_ref[0] + h_ref[1]) * jnp.float32(SCALE)
    h = jnp.clip(hsum, -ABS_MAX, ABS_MAX)
    fe = jnp.clip(f_ref[...] * jnp.float32(SCALE), -ABS_MAX, ABS_MAX)
    feats = jnp.clip(h + fe, -ABS_MAX, ABS_MAX)
    o_ref[...] = lax.dot(feats, r_ref[...],
                         precision=lax.Precision.HIGHEST,
                         preferred_element_type=jnp.float32)


_BR = 1000

_tc_project = pl.pallas_call(
    _tc_body,
    grid=(N // _BR,),
    in_specs=[
        pl.BlockSpec((NC, _BR, D), lambda i: (0, i, 0)),
        pl.BlockSpec((_BR, D), lambda i: (i, 0)),
        pl.BlockSpec((D, D), lambda i: (0, 0)),
    ],
    out_specs=pl.BlockSpec((_BR, D), lambda i: (i, 0)),
    out_shape=jax.ShapeDtypeStruct((N, D), jnp.float32),
)


def kernel(features, edge_index, R):
    src = edge_index[0].astype(jnp.int32)
    dst = edge_index[1].astype(jnp.int32)
    live_pad = TOTAL_ALLOC * CHUNK - E
    src3 = jnp.concatenate([src, jnp.zeros((live_pad,), jnp.int32)])
    dst3 = jnp.concatenate([dst, jnp.full((live_pad,), N, jnp.int32)])
    sd = jnp.stack([src3.reshape(TOTAL_ALLOC, CHUNK),
                    dst3.reshape(TOTAL_ALLOC, CHUNK)], axis=1)
    h2 = _sc_scatter(features, sd)
    r_scaled = R * jnp.float32(1.0 / (math.sqrt(D) * math.sqrt(D)))
    return _tc_project(h2, features, r_scaled)
